# Initial kernel scaffold; baseline (speedup 1.0000x reference)
import functools

import jax
import jax.numpy as jnp
from jax import lax
from jax.experimental import pallas as pl
from jax.experimental.pallas import tpu as pltpu
from jax.experimental.pallas import tpu_sc as plsc

NC = 2
NS = 16
NW = NC * NS
LANES = 16

CH = 128
KSUB = 4
CHS = CH * KSUB


def _deg_body(dst_hbm, ones_hbm, zeros_hbm, out_hbm, idx_v, ones_v, zb_v, deg_sp):
    cid = lax.axis_index("c")
    sid = lax.axis_index("s")
    wid = sid * NC + cid
    npad = out_hbm.shape[1]
    rows_per_sub = npad // NS
    n_sup = dst_hbm.shape[0] // (NW * KSUB)
    base_row = wid * n_sup * KSUB

    pltpu.sync_copy(ones_hbm, ones_v)
    pltpu.sync_copy(zeros_hbm, zb_v)
    pltpu.sync_copy(zb_v, deg_sp.at[pl.ds(sid * rows_per_sub, rows_per_sub)])
    plsc.subcore_barrier()

    def loop(g, carry):
        pltpu.sync_copy(dst_hbm.at[pl.ds(base_row + g * KSUB, KSUB)], idx_v)
        for j in range(KSUB):
            pltpu.sync_copy(ones_v, deg_sp.at[idx_v.at[j]], add=True)
        return carry

    lax.fori_loop(0, n_sup, loop, 0)
    plsc.subcore_barrier()
    pltpu.sync_copy(
        deg_sp.at[pl.ds(sid * rows_per_sub, rows_per_sub)],
        out_hbm.at[cid].at[pl.ds(sid * rows_per_sub, rows_per_sub)],
    )


def _scat_body(src_hbm, dst_hbm, xs_hbm, zeros_hbm, out_hbm,
               sidx_v, didx_v, rows_v, zb_v, acc_sp, sem):
    cid = lax.axis_index("c")
    sid = lax.axis_index("s")
    wid = sid * NC + cid
    npad = out_hbm.shape[1]
    rows_per_sub = npad // NS
    n_sup = src_hbm.shape[0] // (NW * KSUB)
    base_row = wid * n_sup * KSUB

    pltpu.sync_copy(zeros_hbm, zb_v)
    for t in range(rows_per_sub // CH):
        pltpu.sync_copy(zb_v, acc_sp.at[pl.ds(sid * rows_per_sub + t * CH, CH)])
    plsc.subcore_barrier()

    def loop(g, carry):
        pltpu.sync_copy(src_hbm.at[pl.ds(base_row + g * KSUB, KSUB)], sidx_v)
        pltpu.sync_copy(dst_hbm.at[pl.ds(base_row + g * KSUB, KSUB)], didx_v)
        for j in range(KSUB):
            pltpu.async_copy(xs_hbm.at[sidx_v.at[j]],
                             rows_v.at[pl.ds(j * CH, CH)], sem)
        for j in range(KSUB):
            pltpu.async_copy(xs_hbm.at[sidx_v.at[j]],
                             rows_v.at[pl.ds(j * CH, CH)], sem).wait()
        for j in range(KSUB):
            pltpu.sync_copy(rows_v.at[pl.ds(j * CH, CH)],
                            acc_sp.at[didx_v.at[j]], add=True)
        return carry

    lax.fori_loop(0, n_sup, loop, 0)
    plsc.subcore_barrier()
    pltpu.sync_copy(
        acc_sp.at[pl.ds(sid * rows_per_sub, rows_per_sub)],
        out_hbm.at[cid].at[pl.ds(sid * rows_per_sub, rows_per_sub)],
    )


def _prep_body(z_ref, w_ref, deg_ref, xs_ref, dv_ref):
    x = jnp.dot(z_ref[...], w_ref[...], preferred_element_type=jnp.float32)
    deg = deg_ref[0, :, 0:1] + deg_ref[1, :, 0:1] + 1.0
    dinv = lax.rsqrt(deg)
    xs_ref[...] = x * dinv
    dv_ref[...] = jnp.broadcast_to(dinv, dv_ref.shape)


def _h_body(acc_ref, xs_ref, dv_ref, b_ref, h_ref):
    acc = acc_ref[0] + acc_ref[1] + xs_ref[...]
    h_ref[...] = jnp.maximum(acc * dv_ref[...] + b_ref[...], 0.0)


def _mm_body(hi_ref, hj_ref, out_ref):
    out_ref[...] = lax.dot_general(
        hi_ref[...], hj_ref[...], (((1,), (1,)), ((), ())),
        preferred_element_type=jnp.float32)


def kernel(z, edge_index, W, b):
    n, d = z.shape
    e = edge_index.shape[1]

    npad = 10240 if n <= 10240 else ((n + NW * LANES) // (NW * LANES)) * NW * LANES
    ew = ((e + NW * CHS - 1) // (NW * CHS)) * CHS
    epad = ew * NW

    ei = edge_index.astype(jnp.int32)
    pad_node = jnp.int32(n)
    src = jnp.full((epad,), pad_node, jnp.int32)
    src = lax.dynamic_update_slice(src, ei[0], (0,)).reshape(-1, CH)
    dst = jnp.full((epad,), pad_node, jnp.int32)
    dst = lax.dynamic_update_slice(dst, ei[1], (0,)).reshape(-1, CH)

    zp = jnp.zeros((npad, d), jnp.float32).at[:n].set(z.astype(jnp.float32))

    ones16 = jnp.ones((CH, LANES), jnp.float32)
    zeros16 = jnp.zeros((npad // NS, LANES), jnp.float32)
    zeros_d = jnp.zeros((CH, d), jnp.float32)

    mesh = plsc.VectorSubcoreMesh(core_axis_name="c", subcore_axis_name="s",
                                  num_cores=NC, num_subcores=NS)

    deg_parts = pl.kernel(
        _deg_body,
        out_type=jax.ShapeDtypeStruct((NC, npad, LANES), jnp.float32),
        mesh=mesh,
        scratch_types=[
            pltpu.VMEM((KSUB, CH), jnp.int32),
            pltpu.VMEM((CH, LANES), jnp.float32),
            pltpu.VMEM((npad // NS, LANES), jnp.float32),
            pltpu.VMEM_SHARED((npad, LANES), jnp.float32),
        ],
    )(dst, ones16, zeros16)

    blk = 1024
    xs, dv = pl.pallas_call(
        _prep_body,
        grid=(npad // blk,),
        in_specs=[
            pl.BlockSpec((blk, d), lambda i: (i, 0)),
            pl.BlockSpec((d, d), lambda i: (0, 0)),
            pl.BlockSpec((NC, blk, LANES), lambda i: (0, i, 0)),
        ],
        out_specs=[
            pl.BlockSpec((blk, d), lambda i: (i, 0)),
            pl.BlockSpec((blk, d), lambda i: (i, 0)),
        ],
        out_shape=[
            jax.ShapeDtypeStruct((npad, d), jnp.float32),
            jax.ShapeDtypeStruct((npad, d), jnp.float32),
        ],
    )(zp, W, deg_parts)

    acc_parts = pl.kernel(
        _scat_body,
        out_type=jax.ShapeDtypeStruct((NC, npad, d), jnp.float32),
        mesh=mesh,
        scratch_types=[
            pltpu.VMEM((KSUB, CH), jnp.int32),
            pltpu.VMEM((KSUB, CH), jnp.int32),
            pltpu.VMEM((CHS, d), jnp.float32),
            pltpu.VMEM((CH, d), jnp.float32),
            pltpu.VMEM_SHARED((npad, d), jnp.float32),
            pltpu.SemaphoreType.DMA,
        ],
    )(src, dst, xs, zeros_d)

    h = pl.pallas_call(
        _h_body,
        grid=(npad // blk,),
        in_specs=[
            pl.BlockSpec((NC, blk, d), lambda i: (0, i, 0)),
            pl.BlockSpec((blk, d), lambda i: (i, 0)),
            pl.BlockSpec((blk, d), lambda i: (i, 0)),
            pl.BlockSpec((1, d), lambda i: (0, 0)),
        ],
        out_specs=pl.BlockSpec((blk, d), lambda i: (i, 0)),
        out_shape=jax.ShapeDtypeStruct((npad, d), jnp.float32),
    )(acc_parts, xs, dv, b.reshape(1, d))

    bm = bn = 1024
    a_hat = pl.pallas_call(
        _mm_body,
        grid=(npad // bm, npad // bn),
        in_specs=[
            pl.BlockSpec((bm, d), lambda i, j: (i, 0)),
            pl.BlockSpec((bn, d), lambda i, j: (j, 0)),
        ],
        out_specs=pl.BlockSpec((bm, bn), lambda i, j: (i, j)),
        out_shape=jax.ShapeDtypeStruct((n, n), jnp.float32),
    )(h, h)
    return a_hat


# speedup vs baseline: 17.4239x; 17.4239x over previous
import functools

import jax
import jax.numpy as jnp
from jax import lax
from jax.experimental import pallas as pl
from jax.experimental.pallas import tpu as pltpu
from jax.experimental.pallas import tpu_sc as plsc

NC = 2
NS = 16
NW = NC * NS
LANES = 16

CH = 128
KSUB = 4
CHS = CH * KSUB


def _deg_body(dst_hbm, ones_hbm, zeros_hbm, out_hbm, idx_v, ones_v, zb_v, deg_sp):
    cid = lax.axis_index("c")
    sid = lax.axis_index("s")
    wid = sid * NC + cid
    npad = out_hbm.shape[1]
    rows_per_sub = npad // NS
    n_sup = dst_hbm.shape[0] // (NW * KSUB)
    base_row = wid * n_sup * KSUB

    pltpu.sync_copy(ones_hbm, ones_v)
    pltpu.sync_copy(zeros_hbm, zb_v)
    pltpu.sync_copy(zb_v, deg_sp.at[pl.ds(sid * rows_per_sub, rows_per_sub)])
    plsc.subcore_barrier()

    def loop(g, carry):
        pltpu.sync_copy(dst_hbm.at[pl.ds(base_row + g * KSUB, KSUB)], idx_v)
        for j in range(KSUB):
            pltpu.sync_copy(ones_v, deg_sp.at[idx_v.at[j]], add=True)
        return carry

    lax.fori_loop(0, n_sup, loop, 0)
    plsc.subcore_barrier()
    pltpu.sync_copy(
        deg_sp.at[pl.ds(sid * rows_per_sub, rows_per_sub)],
        out_hbm.at[cid].at[pl.ds(sid * rows_per_sub, rows_per_sub)],
    )


def _scat_body(src_hbm, dst_hbm, xs_hbm, zeros_hbm, out_hbm,
               sidx_v, didx_v, rows_v, zb_v, acc_sp, sem):
    cid = lax.axis_index("c")
    sid = lax.axis_index("s")
    wid = sid * NC + cid
    npad = out_hbm.shape[1]
    rows_per_sub = npad // NS
    n_sup = src_hbm.shape[0] // (NW * KSUB)
    base_row = wid * n_sup * KSUB

    pltpu.sync_copy(zeros_hbm, zb_v)
    for t in range(rows_per_sub // CH):
        pltpu.sync_copy(zb_v, acc_sp.at[pl.ds(sid * rows_per_sub + t * CH, CH)])
    plsc.subcore_barrier()

    def loop(g, carry):
        pltpu.sync_copy(src_hbm.at[pl.ds(base_row + g * KSUB, KSUB)], sidx_v)
        pltpu.sync_copy(dst_hbm.at[pl.ds(base_row + g * KSUB, KSUB)], didx_v)
        descs = [pltpu.async_copy(xs_hbm.at[sidx_v.at[j]],
                                  rows_v.at[pl.ds(j * CH, CH)], sem)
                 for j in range(KSUB)]
        for dsc in descs:
            dsc.wait()
        for j in range(KSUB):
            pltpu.sync_copy(rows_v.at[pl.ds(j * CH, CH)],
                            acc_sp.at[didx_v.at[j]], add=True)
        return carry

    lax.fori_loop(0, n_sup, loop, 0)
    plsc.subcore_barrier()
    pltpu.sync_copy(
        acc_sp.at[pl.ds(sid * rows_per_sub, rows_per_sub)],
        out_hbm.at[cid].at[pl.ds(sid * rows_per_sub, rows_per_sub)],
    )


def _prep_body(z_ref, w_ref, deg_ref, xs_ref, dv_ref):
    x = jnp.dot(z_ref[...], w_ref[...], preferred_element_type=jnp.float32)
    deg = deg_ref[0, :, 0:1] + deg_ref[1, :, 0:1] + 1.0
    dinv = lax.rsqrt(deg)
    xs_ref[...] = x * dinv
    dv_ref[...] = jnp.broadcast_to(dinv, dv_ref.shape)


def _h_body(acc_ref, xs_ref, dv_ref, b_ref, h_ref):
    acc = acc_ref[0] + acc_ref[1] + xs_ref[...]
    h_ref[...] = jnp.maximum(acc * dv_ref[...] + b_ref[...], 0.0)


def _mm_body(hi_ref, hj_ref, out_ref):
    out_ref[...] = lax.dot_general(
        hi_ref[...], hj_ref[...], (((1,), (1,)), ((), ())),
        preferred_element_type=jnp.float32)


def kernel(z, edge_index, W, b):
    n, d = z.shape
    e = edge_index.shape[1]

    npad = 10240 if n <= 10240 else ((n + NW * LANES) // (NW * LANES)) * NW * LANES
    ew = ((e + NW * CHS - 1) // (NW * CHS)) * CHS
    epad = ew * NW

    ei = edge_index.astype(jnp.int32)
    pad_node = jnp.int32(n)
    src = jnp.full((epad,), pad_node, jnp.int32)
    src = lax.dynamic_update_slice(src, ei[0], (0,)).reshape(-1, CH)
    dst = jnp.full((epad,), pad_node, jnp.int32)
    dst = lax.dynamic_update_slice(dst, ei[1], (0,)).reshape(-1, CH)

    zp = jnp.zeros((npad, d), jnp.float32).at[:n].set(z.astype(jnp.float32))

    ones16 = jnp.ones((CH, LANES), jnp.float32)
    zeros16 = jnp.zeros((npad // NS, LANES), jnp.float32)
    zeros_d = jnp.zeros((CH, d), jnp.float32)

    mesh = plsc.VectorSubcoreMesh(core_axis_name="c", subcore_axis_name="s",
                                  num_cores=NC, num_subcores=NS)
    sc_params = pltpu.CompilerParams(use_tc_tiling_on_sc=False)

    deg_parts = pl.kernel(
        _deg_body,
        out_type=jax.ShapeDtypeStruct((NC, npad, LANES), jnp.float32),
        mesh=mesh,
        scratch_types=[
            pltpu.VMEM((KSUB, CH), jnp.int32),
            pltpu.VMEM((CH, LANES), jnp.float32),
            pltpu.VMEM((npad // NS, LANES), jnp.float32),
            pltpu.VMEM_SHARED((npad, LANES), jnp.float32),
        ],
        compiler_params=sc_params,
    )(dst, ones16, zeros16)

    blk = 1024
    xs, dv = pl.pallas_call(
        _prep_body,
        grid=(npad // blk,),
        in_specs=[
            pl.BlockSpec((blk, d), lambda i: (i, 0)),
            pl.BlockSpec((d, d), lambda i: (0, 0)),
            pl.BlockSpec((NC, blk, LANES), lambda i: (0, i, 0)),
        ],
        out_specs=[
            pl.BlockSpec((blk, d), lambda i: (i, 0)),
            pl.BlockSpec((blk, d), lambda i: (i, 0)),
        ],
        out_shape=[
            jax.ShapeDtypeStruct((npad, d), jnp.float32),
            jax.ShapeDtypeStruct((npad, d), jnp.float32),
        ],
    )(zp, W, deg_parts)

    acc_parts = pl.kernel(
        _scat_body,
        out_type=jax.ShapeDtypeStruct((NC, npad, d), jnp.float32),
        mesh=mesh,
        scratch_types=[
            pltpu.VMEM((KSUB, CH), jnp.int32),
            pltpu.VMEM((KSUB, CH), jnp.int32),
            pltpu.VMEM((CHS, d), jnp.float32),
            pltpu.VMEM((CH, d), jnp.float32),
            pltpu.VMEM_SHARED((npad, d), jnp.float32),
            pltpu.SemaphoreType.DMA,
        ],
        compiler_params=sc_params,
    )(src, dst, xs, zeros_d)

    h = pl.pallas_call(
        _h_body,
        grid=(npad // blk,),
        in_specs=[
            pl.BlockSpec((NC, blk, d), lambda i: (0, i, 0)),
            pl.BlockSpec((blk, d), lambda i: (i, 0)),
            pl.BlockSpec((blk, d), lambda i: (i, 0)),
            pl.BlockSpec((1, d), lambda i: (0, 0)),
        ],
        out_specs=pl.BlockSpec((blk, d), lambda i: (i, 0)),
        out_shape=jax.ShapeDtypeStruct((npad, d), jnp.float32),
    )(acc_parts, xs, dv, b.reshape(1, d))

    bm = bn = 1024
    a_hat = pl.pallas_call(
        _mm_body,
        grid=(npad // bm, npad // bn),
        in_specs=[
            pl.BlockSpec((bm, d), lambda i, j: (i, 0)),
            pl.BlockSpec((bn, d), lambda i, j: (j, 0)),
        ],
        out_specs=pl.BlockSpec((bm, bn), lambda i, j: (i, j)),
        out_shape=jax.ShapeDtypeStruct((n, n), jnp.float32),
    )(h, h)
    return a_hat


# speedup vs baseline: 19.2343x; 1.1039x over previous
import functools

import jax
import jax.numpy as jnp
from jax import lax
from jax.experimental import pallas as pl
from jax.experimental.pallas import tpu as pltpu
from jax.experimental.pallas import tpu_sc as plsc

NC = 2
NS = 16
NW = NC * NS
LANES = 16

CH = 128
KSUB = 4
CHS = CH * KSUB


def _deg_body(dst_hbm, ones_hbm, zeros_hbm, out_hbm, idx_v, ones_v, zb_v, deg_sp):
    cid = lax.axis_index("c")
    sid = lax.axis_index("s")
    wid = sid * NC + cid
    npad = out_hbm.shape[1]
    rows_per_sub = npad // NS
    n_sup = dst_hbm.shape[0] // (NW * KSUB)
    base_row = wid * n_sup * KSUB

    pltpu.sync_copy(ones_hbm, ones_v)
    pltpu.sync_copy(zeros_hbm, zb_v)
    pltpu.sync_copy(zb_v, deg_sp.at[pl.ds(sid * rows_per_sub, rows_per_sub)])
    plsc.subcore_barrier()

    def loop(g, carry):
        pltpu.sync_copy(dst_hbm.at[pl.ds(base_row + g * KSUB, KSUB)], idx_v)
        for j in range(KSUB):
            pltpu.sync_copy(ones_v, deg_sp.at[idx_v.at[j]], add=True)
        return carry

    lax.fori_loop(0, n_sup, loop, 0)
    plsc.subcore_barrier()
    pltpu.sync_copy(
        deg_sp.at[pl.ds(sid * rows_per_sub, rows_per_sub)],
        out_hbm.at[cid].at[pl.ds(sid * rows_per_sub, rows_per_sub)],
    )


def _scat_body(src_hbm, dst_hbm, xs_hbm, zeros_hbm, out_hbm,
               sidx_v, didx_v, rows_v, zb_v, acc_sp, gsem0, gsem1):
    cid = lax.axis_index("c")
    sid = lax.axis_index("s")
    wid = sid * NC + cid
    npad = out_hbm.shape[1]
    rows_per_sub = npad // NS
    n_sup = src_hbm.shape[0] // (NW * KSUB)
    base_row = wid * n_sup * KSUB
    gsems = (gsem0, gsem1)

    pltpu.sync_copy(zeros_hbm, zb_v)
    for t in range(rows_per_sub // CH):
        pltpu.sync_copy(zb_v, acc_sp.at[pl.ds(sid * rows_per_sub + t * CH, CH)])
    plsc.subcore_barrier()

    def fire(b, g):
        pltpu.sync_copy(src_hbm.at[pl.ds(base_row + g * KSUB, KSUB)],
                        sidx_v.at[b])
        pltpu.sync_copy(dst_hbm.at[pl.ds(base_row + g * KSUB, KSUB)],
                        didx_v.at[b])
        for j in range(KSUB):
            pltpu.async_copy(xs_hbm.at[sidx_v.at[b].at[j]],
                             rows_v.at[b].at[pl.ds(j * CH, CH)], gsems[b])

    def drain_scatter(b):
        for j in range(KSUB):
            pltpu.make_async_copy(xs_hbm.at[sidx_v.at[b].at[j]],
                                  rows_v.at[b].at[pl.ds(j * CH, CH)],
                                  gsems[b]).wait()
        for j in range(KSUB):
            pltpu.sync_copy(rows_v.at[b].at[pl.ds(j * CH, CH)],
                            acc_sp.at[didx_v.at[b].at[j]], add=True)

    fire(0, 0)

    def loop(g2, carry):
        fire(1, 2 * g2 + 1)
        drain_scatter(0)

        @pl.when(2 * g2 + 2 < n_sup)
        def _():
            fire(0, 2 * g2 + 2)

        drain_scatter(1)
        return carry

    lax.fori_loop(0, n_sup // 2, loop, 0)
    plsc.subcore_barrier()
    pltpu.sync_copy(
        acc_sp.at[pl.ds(sid * rows_per_sub, rows_per_sub)],
        out_hbm.at[cid].at[pl.ds(sid * rows_per_sub, rows_per_sub)],
    )


def _prep_body(z_ref, w_ref, deg_ref, xs_ref, dv_ref):
    x = jnp.dot(z_ref[...], w_ref[...], preferred_element_type=jnp.float32)
    deg = deg_ref[0, :, 0:1] + deg_ref[1, :, 0:1] + 1.0
    dinv = lax.rsqrt(deg)
    xs_ref[...] = x * dinv
    dv_ref[...] = jnp.broadcast_to(dinv, dv_ref.shape)


def _h_body(acc_ref, xs_ref, dv_ref, b_ref, h_ref):
    acc = acc_ref[0] + acc_ref[1] + xs_ref[...]
    h_ref[...] = jnp.maximum(acc * dv_ref[...] + b_ref[...], 0.0)


def _mm_body(hi_ref, hj_ref, out_ref):
    out_ref[...] = lax.dot_general(
        hi_ref[...], hj_ref[...], (((1,), (1,)), ((), ())),
        preferred_element_type=jnp.float32)


def kernel(z, edge_index, W, b):
    n, d = z.shape
    e = edge_index.shape[1]

    npad = 10240 if n <= 10240 else ((n + NW * LANES) // (NW * LANES)) * NW * LANES
    ew = ((e + NW * CHS - 1) // (NW * CHS)) * CHS
    epad = ew * NW

    ei = edge_index.astype(jnp.int32)
    pad_node = jnp.int32(n)
    src = jnp.full((epad,), pad_node, jnp.int32)
    src = lax.dynamic_update_slice(src, ei[0], (0,)).reshape(-1, CH)
    dst = jnp.full((epad,), pad_node, jnp.int32)
    dst = lax.dynamic_update_slice(dst, ei[1], (0,)).reshape(-1, CH)

    zp = jnp.zeros((npad, d), jnp.float32).at[:n].set(z.astype(jnp.float32))

    ones16 = jnp.ones((CH, LANES), jnp.float32)
    zeros16 = jnp.zeros((npad // NS, LANES), jnp.float32)
    zeros_d = jnp.zeros((CH, d), jnp.float32)

    mesh = plsc.VectorSubcoreMesh(core_axis_name="c", subcore_axis_name="s",
                                  num_cores=NC, num_subcores=NS)
    sc_params = pltpu.CompilerParams(use_tc_tiling_on_sc=False)

    deg_parts = pl.kernel(
        _deg_body,
        out_type=jax.ShapeDtypeStruct((NC, npad, LANES), jnp.float32),
        mesh=mesh,
        scratch_types=[
            pltpu.VMEM((KSUB, CH), jnp.int32),
            pltpu.VMEM((CH, LANES), jnp.float32),
            pltpu.VMEM((npad // NS, LANES), jnp.float32),
            pltpu.VMEM_SHARED((npad, LANES), jnp.float32),
        ],
        compiler_params=sc_params,
    )(dst, ones16, zeros16)

    blk = 1024
    xs, dv = pl.pallas_call(
        _prep_body,
        grid=(npad // blk,),
        in_specs=[
            pl.BlockSpec((blk, d), lambda i: (i, 0)),
            pl.BlockSpec((d, d), lambda i: (0, 0)),
            pl.BlockSpec((NC, blk, LANES), lambda i: (0, i, 0)),
        ],
        out_specs=[
            pl.BlockSpec((blk, d), lambda i: (i, 0)),
            pl.BlockSpec((blk, d), lambda i: (i, 0)),
        ],
        out_shape=[
            jax.ShapeDtypeStruct((npad, d), jnp.float32),
            jax.ShapeDtypeStruct((npad, d), jnp.float32),
        ],
    )(zp, W, deg_parts)

    acc_parts = pl.kernel(
        _scat_body,
        out_type=jax.ShapeDtypeStruct((NC, npad, d), jnp.float32),
        mesh=mesh,
        scratch_types=[
            pltpu.VMEM((2, KSUB, CH), jnp.int32),
            pltpu.VMEM((2, KSUB, CH), jnp.int32),
            pltpu.VMEM((2, CHS, d), jnp.float32),
            pltpu.VMEM((CH, d), jnp.float32),
            pltpu.VMEM_SHARED((npad, d), jnp.float32),
            pltpu.SemaphoreType.DMA,
            pltpu.SemaphoreType.DMA,
        ],
        compiler_params=sc_params,
    )(src, dst, xs, zeros_d)

    h = pl.pallas_call(
        _h_body,
        grid=(npad // blk,),
        in_specs=[
            pl.BlockSpec((NC, blk, d), lambda i: (0, i, 0)),
            pl.BlockSpec((blk, d), lambda i: (i, 0)),
            pl.BlockSpec((blk, d), lambda i: (i, 0)),
            pl.BlockSpec((1, d), lambda i: (0, 0)),
        ],
        out_specs=pl.BlockSpec((blk, d), lambda i: (i, 0)),
        out_shape=jax.ShapeDtypeStruct((npad, d), jnp.float32),
    )(acc_parts, xs, dv, b.reshape(1, d))

    bm = bn = 1024
    a_hat = pl.pallas_call(
        _mm_body,
        grid=(npad // bm, npad // bn),
        in_specs=[
            pl.BlockSpec((bm, d), lambda i, j: (i, 0)),
            pl.BlockSpec((bn, d), lambda i, j: (j, 0)),
        ],
        out_specs=pl.BlockSpec((bm, bn), lambda i, j: (i, j)),
        out_shape=jax.ShapeDtypeStruct((n, n), jnp.float32),
    )(h, h)
    return a_hat
